# baseline (speedup 1.0000x reference)
import jax
import jax.numpy as jnp
from jax.experimental import pallas as pl
from jax.experimental.pallas import tpu as pltpu


_BE = 64
_NBUF = 6


def _make_body(E, K, O, nblk):
    def body(ea_ref, w_ref, b_ref, agg_ref, ee_ref, scratch, sems):
        b = b_ref[...]
        w = w_ref[...]

        def step(i, carry):
            j = jax.lax.rem(i, _NBUF)

            @pl.when(i >= _NBUF)
            def _wait_prev():
                pltpu.make_async_copy(
                    scratch.at[j],
                    agg_ref.at[pl.ds((i - _NBUF) * _BE, _BE), :, :],
                    sems.at[j],
                ).wait()

            ea_blk = ea_ref[pl.ds(i * _BE, _BE), :]
            ee_blk = jnp.sum(ea_blk * w, axis=1, keepdims=True)
            ee_ref[pl.ds(i * _BE, _BE), :] = ee_blk
            val = jnp.maximum(ee_blk[:, :, None] + b, 0.0)
            scratch[j] = jnp.broadcast_to(val, (_BE, K, O))
            pltpu.make_async_copy(
                scratch.at[j],
                agg_ref.at[pl.ds(i * _BE, _BE), :, :],
                sems.at[j],
            ).start()
            return carry

        jax.lax.fori_loop(0, nblk, step, 0)

        def drain(i, carry):
            j = jax.lax.rem(i, _NBUF)
            pltpu.make_async_copy(
                scratch.at[j],
                agg_ref.at[pl.ds(i * _BE, _BE), :, :],
                sems.at[j],
            ).wait()
            return carry

        jax.lax.fori_loop(nblk - _NBUF, nblk, drain, 0)

    return body


def kernel(x, edge_index, edge_attr, W, a, bias, edge_embedding_weight):
    E, _, D = edge_attr.shape
    O = bias.shape[0]
    K = a.shape[1]
    nblk = E // _BE

    ea2 = edge_attr.reshape(E, D)
    w_row = edge_embedding_weight.reshape(1, D)
    b3 = bias.reshape(1, 1, O)

    agg, ee2 = pl.pallas_call(
        _make_body(E, K, O, nblk),
        in_specs=[
            pl.BlockSpec(memory_space=pltpu.MemorySpace.VMEM),
            pl.BlockSpec(memory_space=pltpu.MemorySpace.VMEM),
            pl.BlockSpec(memory_space=pltpu.MemorySpace.VMEM),
        ],
        out_specs=[
            pl.BlockSpec(memory_space=pl.ANY),
            pl.BlockSpec(memory_space=pltpu.MemorySpace.VMEM),
        ],
        out_shape=[
            jax.ShapeDtypeStruct((E, K, O), jnp.float32),
            jax.ShapeDtypeStruct((E, 1), jnp.float32),
        ],
        scratch_shapes=[
            pltpu.VMEM((_NBUF, _BE, K, O), jnp.float32),
            pltpu.SemaphoreType.DMA((_NBUF,)),
        ],
    )(ea2, w_row, b3)

    edge_embeddings = ee2.reshape(E, 1, 1)
    return (agg, edge_embeddings)
